# Initial kernel scaffold; baseline (speedup 1.0000x reference)
import jax
import jax.numpy as jnp
from jax.experimental import pallas as pl


def kernel(input, W, b, train_features):
    raise NotImplementedError("write your pallas kernel here")


# speedup vs baseline: 4.8337x; 4.8337x over previous
import jax
import jax.numpy as jnp
from jax.experimental import pallas as pl
from jax.experimental.pallas import tpu as pltpu

_B = 1024
_DIN = 64
_DF = 16
_K = 100000
_TK = 2048
_NK = (_K + _TK - 1) // _TK


def _knn_body(x_ref, w_ref, b_ref, t_ref, o_ref, ta_ref, m1_ref, m2_ref):
    k = pl.program_id(0)

    @pl.when(k == 0)
    def _init():
        test = jnp.dot(x_ref[...], w_ref[...],
                       preferred_element_type=jnp.float32) + b_ref[...]
        ta_ref[...] = jnp.concatenate(
            [-2.0 * test, jnp.ones((_B, 1), jnp.float32)], axis=1)
        inf = jnp.full((_B, 1), jnp.inf, jnp.float32)
        m1_ref[...] = inf
        m2_ref[...] = inf

    tt = t_ref[...]
    t2 = jnp.sum(tt * tt, axis=1, keepdims=True)
    taug = jnp.concatenate([tt, t2], axis=1)
    s = jax.lax.dot_general(
        ta_ref[...], taug, (((1,), (1,)), ((), ())),
        preferred_element_type=jnp.float32)

    col = jax.lax.broadcasted_iota(jnp.int32, (_B, _TK), 1)
    s = jnp.where(col + k * _TK < _K, s, jnp.inf)

    min1 = jnp.min(s, axis=1, keepdims=True)
    idx1 = jnp.min(jnp.where(s == min1, col, _TK), axis=1, keepdims=True)
    min2 = jnp.min(jnp.where(col == idx1, jnp.inf, s), axis=1, keepdims=True)

    m1, m2 = m1_ref[...], m2_ref[...]
    lo = jnp.minimum(m1, min1)
    hi = jnp.minimum(jnp.maximum(m1, min1), jnp.minimum(m2, min2))
    m1_ref[...] = lo
    m2_ref[...] = hi

    @pl.when(k == _NK - 1)
    def _fin():
        ta = ta_ref[...]
        q2 = 0.25 * jnp.sum(ta[:, :_DF] * ta[:, :_DF], axis=1, keepdims=True)
        o_ref[...] = lo + hi + 2.0 * q2


def kernel(input, W, b, train_features):
    out = pl.pallas_call(
        _knn_body,
        grid=(_NK,),
        in_specs=[
            pl.BlockSpec((_B, _DIN), lambda k: (0, 0)),
            pl.BlockSpec((_DIN, _DF), lambda k: (0, 0)),
            pl.BlockSpec((1, _DF), lambda k: (0, 0)),
            pl.BlockSpec((_TK, _DF), lambda k: (k, 0)),
        ],
        out_specs=pl.BlockSpec((_B, 1), lambda k: (0, 0)),
        out_shape=jax.ShapeDtypeStruct((_B, 1), jnp.float32),
        scratch_shapes=[
            pltpu.VMEM((_B, _DF + 1), jnp.float32),
            pltpu.VMEM((_B, 1), jnp.float32),
            pltpu.VMEM((_B, 1), jnp.float32),
        ],
    )(input, W, b.reshape(1, _DF), train_features)
    return out.reshape(_B)


# speedup vs baseline: 4.8712x; 1.0077x over previous
import jax
import jax.numpy as jnp
from jax.experimental import pallas as pl
from jax.experimental.pallas import tpu as pltpu

_B = 1024
_DIN = 64
_DF = 16
_K = 100000
_TK = 2048
_NK = (_K + _TK - 1) // _TK
_BIG = 1e9


def _top2_tile(s):
    w = s.shape[1] // 2
    lo = jnp.minimum(s[:, :w], s[:, w:])
    hi = jnp.maximum(s[:, :w], s[:, w:])
    while w > 1:
        w //= 2
        la, lb = lo[:, :w], lo[:, w:]
        ha, hb = hi[:, :w], hi[:, w:]
        lo = jnp.minimum(la, lb)
        hi = jnp.minimum(jnp.maximum(la, lb), jnp.minimum(ha, hb))
    return lo, hi


def _knn_body(x_ref, w_ref, b_ref, t_ref, o_ref, ta_ref, m1_ref, m2_ref):
    k = pl.program_id(0)

    @pl.when(k == 0)
    def _init():
        test = jnp.dot(x_ref[...], w_ref[...],
                       preferred_element_type=jnp.float32) + b_ref[...]
        ta_ref[...] = jnp.concatenate(
            [-2.0 * test, jnp.ones((_B, 1), jnp.float32)], axis=1)
        inf = jnp.full((_B, 1), jnp.inf, jnp.float32)
        m1_ref[...] = inf
        m2_ref[...] = inf

    row = jax.lax.broadcasted_iota(jnp.int32, (_TK, 1), 0)
    ok = row + k * _TK < _K
    tt = jnp.where(ok, t_ref[...], 0.0)
    t2 = jnp.sum(tt * tt, axis=1, keepdims=True)
    t2 = jnp.where(ok, t2, _BIG)
    taug = jnp.concatenate([tt, t2], axis=1)
    s = jax.lax.dot_general(
        ta_ref[...], taug, (((1,), (1,)), ((), ())),
        preferred_element_type=jnp.float32)

    min1, min2 = _top2_tile(s)

    m1, m2 = m1_ref[...], m2_ref[...]
    lo = jnp.minimum(m1, min1)
    hi = jnp.minimum(jnp.maximum(m1, min1), jnp.minimum(m2, min2))
    m1_ref[...] = lo
    m2_ref[...] = hi

    @pl.when(k == _NK - 1)
    def _fin():
        ta = ta_ref[...]
        q2 = 0.25 * jnp.sum(ta[:, :_DF] * ta[:, :_DF], axis=1, keepdims=True)
        o_ref[...] = lo + hi + 2.0 * q2


def kernel(input, W, b, train_features):
    out = pl.pallas_call(
        _knn_body,
        grid=(_NK,),
        in_specs=[
            pl.BlockSpec((_B, _DIN), lambda k: (0, 0)),
            pl.BlockSpec((_DIN, _DF), lambda k: (0, 0)),
            pl.BlockSpec((1, _DF), lambda k: (0, 0)),
            pl.BlockSpec((_TK, _DF), lambda k: (k, 0)),
        ],
        out_specs=pl.BlockSpec((_B, 1), lambda k: (0, 0)),
        out_shape=jax.ShapeDtypeStruct((_B, 1), jnp.float32),
        scratch_shapes=[
            pltpu.VMEM((_B, _DF + 1), jnp.float32),
            pltpu.VMEM((_B, 1), jnp.float32),
            pltpu.VMEM((_B, 1), jnp.float32),
        ],
    )(input, W, b.reshape(1, _DF), train_features)
    return out.reshape(_B)


# speedup vs baseline: 9.7969x; 2.0112x over previous
import jax
import jax.numpy as jnp
from jax.experimental import pallas as pl
from jax.experimental.pallas import tpu as pltpu

_B = 1024
_DIN = 64
_DF = 16
_K = 100000
_TK = 2048
_NK = (_K + _TK - 1) // _TK
_RB = 128
_W = 128
_BIG = 1e9


def _knn_body(x_ref, w_ref, b_ref, t_ref, o_ref, ta_ref, m1_ref, m2_ref):
    k = pl.program_id(0)

    @pl.when(k == 0)
    def _init():
        test = jnp.dot(x_ref[...], w_ref[...],
                       preferred_element_type=jnp.float32) + b_ref[...]
        ta_ref[...] = jnp.concatenate(
            [-2.0 * test, jnp.ones((_B, 1), jnp.float32)], axis=1)
        inf = jnp.full((_B, _W), jnp.inf, jnp.float32)
        m1_ref[...] = inf
        m2_ref[...] = inf

    row = jax.lax.broadcasted_iota(jnp.int32, (_TK, 1), 0)
    ok = row + k * _TK < _K
    tt = jnp.where(ok, t_ref[...], 0.0)
    t2 = jnp.sum(tt * tt, axis=1, keepdims=True)
    t2 = jnp.where(ok, t2, _BIG)
    taug = jnp.concatenate([tt, t2], axis=1)
    s = jax.lax.dot_general(
        ta_ref[...], taug, (((1,), (1,)), ((), ())),
        preferred_element_type=jnp.float32)

    for rb in range(_B // _RB):
        r = pl.ds(rb * _RB, _RB)
        lo = m1_ref[r, :]
        hi = m2_ref[r, :]
        for c in range(_TK // _W):
            v = s[rb * _RB:(rb + 1) * _RB, c * _W:(c + 1) * _W]
            nlo = jnp.minimum(lo, v)
            hi = jnp.minimum(hi, jnp.maximum(lo, v))
            lo = nlo
        m1_ref[r, :] = lo
        m2_ref[r, :] = hi

    @pl.when(k == _NK - 1)
    def _fin():
        lo = m1_ref[...]
        hi = m2_ref[...]
        w = _W
        while w > 1:
            w //= 2
            la, lb = lo[:, :w], lo[:, w:]
            ha, hb = hi[:, :w], hi[:, w:]
            lo = jnp.minimum(la, lb)
            hi = jnp.minimum(jnp.maximum(la, lb), jnp.minimum(ha, hb))
        ta = ta_ref[...]
        q2 = 0.25 * jnp.sum(ta[:, :_DF] * ta[:, :_DF], axis=1, keepdims=True)
        o_ref[...] = lo + hi + 2.0 * q2


def kernel(input, W, b, train_features):
    out = pl.pallas_call(
        _knn_body,
        grid=(_NK,),
        in_specs=[
            pl.BlockSpec((_B, _DIN), lambda k: (0, 0)),
            pl.BlockSpec((_DIN, _DF), lambda k: (0, 0)),
            pl.BlockSpec((1, _DF), lambda k: (0, 0)),
            pl.BlockSpec((_TK, _DF), lambda k: (k, 0)),
        ],
        out_specs=pl.BlockSpec((_B, 1), lambda k: (0, 0)),
        out_shape=jax.ShapeDtypeStruct((_B, 1), jnp.float32),
        scratch_shapes=[
            pltpu.VMEM((_B, _DF + 1), jnp.float32),
            pltpu.VMEM((_B, _W), jnp.float32),
            pltpu.VMEM((_B, _W), jnp.float32),
        ],
    )(input, W, b.reshape(1, _DF), train_features)
    return out.reshape(_B)
